# baseline (speedup 1.0000x reference)
import functools

import jax
import jax.numpy as jnp
from jax import lax
from jax.experimental import pallas as pl
from jax.experimental.pallas import tpu as pltpu
from jax.experimental.pallas import tpu_sc as plsc

N_TOK = 9216
N_EMB = 8192
DIM = 256
BT = 256
NBLK = N_TOK // BT
EPS = 1e-12
_CHUNK = 2736

_SC_NW = 32
_B_PER_W = N_TOK // _SC_NW


def _norm_rows(v):
    n = jnp.sqrt(jnp.sum(jnp.abs(v) ** 2, axis=1, keepdims=True))
    return v / jnp.maximum(n, EPS)


def _dist_body(x_ref, cb_ref, xn_ref, wn_ref, idx_ref, wn2_ref, wnbf_ref):
    i = pl.program_id(0)

    @pl.when(i == 0)
    def _():
        wn = _norm_rows(cb_ref[...])
        wn_ref[...] = wn
        wn2_ref[...] = jnp.sum(wn ** 2, axis=1)[None, :]
        wnbf_ref[...] = wn.astype(jnp.bfloat16)

    xn = _norm_rows(x_ref[...])
    xn_ref[...] = xn
    xn2 = jnp.sum(xn ** 2, axis=1, keepdims=True)
    dots = lax.dot_general(xn.astype(jnp.bfloat16), wnbf_ref[...],
                           (((1,), (1,)), ((), ())),
                           preferred_element_type=jnp.float32)
    dist = (xn2 + wn2_ref[...]) - 2.0 * dots
    acc = jnp.full((dist.shape[0], 1), jnp.inf, jnp.float32)
    idx = jnp.zeros((dist.shape[0], 1), jnp.int32)
    col = lax.broadcasted_iota(jnp.int32, dist.shape, 1)
    inf = jnp.float32(jnp.inf)
    for lo in range(0, N_EMB, _CHUNK):
        hi = min(lo + _CHUNK, N_EMB)
        dw = jnp.where((col >= lo) & (col < hi), dist, inf)
        m = jnp.min(dw, axis=1, keepdims=True)
        i = jnp.min(jnp.where(dw == m, col, jnp.int32(N_EMB)),
                    axis=1, keepdims=True)
        take = m < acc
        idx = jnp.where(take, i, idx)
        acc = jnp.where(take, m.astype(jnp.bfloat16).astype(jnp.float32), acc)
    idx_ref[0, 0, :] = idx[:, 0]


def _st_body(xn_ref, q_ref, qst_ref, loss_ref, acc_ref):
    i = pl.program_id(0)
    xn = xn_ref[...]
    d = q_ref[...] - xn
    qst_ref[...] = xn + d

    @pl.when(i == 0)
    def _():
        acc_ref[0] = 0.0

    acc_ref[0] += jnp.sum(d * d)

    @pl.when(i == NBLK - 1)
    def _():
        loss_ref[...] = jnp.reshape(
            acc_ref[0] * jnp.float32(1.25 / (N_TOK * DIM)), (1, 1))


@functools.cache
def _make_sc_gather():
    @functools.partial(
        pl.kernel,
        out_type=jax.ShapeDtypeStruct((N_TOK, DIM), jnp.float32),
        mesh=plsc.VectorSubcoreMesh(core_axis_name="c", subcore_axis_name="s"),
        scratch_types=[
            pltpu.VMEM((_B_PER_W,), jnp.int32),
            pltpu.VMEM((_B_PER_W, DIM), jnp.float32),
            pltpu.SemaphoreType.DMA,
        ],
    )
    def _sc_gather(table_hbm, idx_hbm, out_hbm, idx_v, rows_v, sem):
        wid = lax.axis_index("s") * 2 + lax.axis_index("c")
        base = wid * _B_PER_W
        pltpu.sync_copy(idx_hbm.at[pl.ds(base, _B_PER_W)], idx_v)
        pltpu.async_copy(table_hbm.at[idx_v], rows_v, sem).wait()
        pltpu.sync_copy(rows_v, out_hbm.at[pl.ds(base, _B_PER_W)])

    return _sc_gather


def kernel(x, codebook):
    xn, wn, idx3 = pl.pallas_call(
        _dist_body,
        grid=(NBLK,),
        in_specs=[
            pl.BlockSpec((BT, DIM), lambda i: (i, 0)),
            pl.BlockSpec((N_EMB, DIM), lambda i: (0, 0)),
        ],
        out_specs=[
            pl.BlockSpec((BT, DIM), lambda i: (i, 0)),
            pl.BlockSpec((N_EMB, DIM), lambda i: (0, 0)),
            pl.BlockSpec((1, 1, BT), lambda i: (i, 0, 0)),
        ],
        out_shape=[
            jax.ShapeDtypeStruct((N_TOK, DIM), jnp.float32),
            jax.ShapeDtypeStruct((N_EMB, DIM), jnp.float32),
            jax.ShapeDtypeStruct((NBLK, 1, BT), jnp.int32),
        ],
        scratch_shapes=[pltpu.VMEM((1, N_EMB), jnp.float32),
                        pltpu.VMEM((N_EMB, DIM), jnp.bfloat16)],
    )(x, codebook)
    idx = idx3.reshape(N_TOK)
    q = _make_sc_gather()(wn, idx)
    qst, loss = pl.pallas_call(
        _st_body,
        grid=(NBLK,),
        in_specs=[
            pl.BlockSpec((BT, DIM), lambda i: (i, 0)),
            pl.BlockSpec((BT, DIM), lambda i: (i, 0)),
        ],
        out_specs=[
            pl.BlockSpec((BT, DIM), lambda i: (i, 0)),
            pl.BlockSpec((1, 1), lambda i: (0, 0)),
        ],
        out_shape=[
            jax.ShapeDtypeStruct((N_TOK, DIM), jnp.float32),
            jax.ShapeDtypeStruct((1, 1), jnp.float32),
        ],
        scratch_shapes=[pltpu.SMEM((1,), jnp.float32)],
    )(xn, q)
    return qst, loss[0, 0], idx


# speedup vs baseline: 1.2838x; 1.2838x over previous
import functools

import jax
import jax.numpy as jnp
from jax import lax
from jax.experimental import pallas as pl
from jax.experimental.pallas import tpu as pltpu
from jax.experimental.pallas import tpu_sc as plsc

N_TOK = 9216
N_EMB = 8192
DIM = 256
BT = 256
NBLK = N_TOK // BT
EPS = 1e-12
_CHUNK = 2736

_SC_NW = 32
_B_PER_W = N_TOK // _SC_NW


def _norm_rows(v):
    n = jnp.sqrt(jnp.sum(jnp.abs(v) ** 2, axis=1, keepdims=True))
    return v / jnp.maximum(n, EPS)


def _dist_body(x_ref, cb_ref, xn_ref, wn_ref, idx_ref, wn2_ref, wnbf_ref):
    i = pl.program_id(0)

    @pl.when(i == 0)
    def _():
        wn = _norm_rows(cb_ref[...])
        wn_ref[...] = wn
        wn2_ref[...] = jnp.sum(wn ** 2, axis=1)[None, :]
        wnbf_ref[...] = wn.astype(jnp.bfloat16)

    xn = _norm_rows(x_ref[...])
    xn_ref[...] = xn
    xn2 = jnp.sum(xn ** 2, axis=1, keepdims=True)
    bt = xn.shape[0]
    xm2b = (-2.0 * xn).astype(jnp.bfloat16)
    inf = jnp.float32(jnp.inf)
    big = jnp.int32(N_EMB)
    lane = lax.broadcasted_iota(jnp.int32, (bt, 128), 1)

    def dchunk(lo, hi):
        dots = lax.dot_general(xm2b, wnbf_ref[lo:hi, :],
                               (((1,), (1,)), ((), ())),
                               preferred_element_type=jnp.float32)
        return (xn2 + wn2_ref[:, lo:hi]) + dots

    c0 = dchunk(0, 2816)
    c1 = dchunk(2816, 5504)
    c2 = dchunk(5504, 8192)

    t21 = c0[:, 2688:2816]
    t42 = c1[:, 2560:2688]
    win_tiles = (
        [(c0[:, t * 128:(t + 1) * 128], t * 128) for t in range(21)]
        + [(jnp.where(lane < 48, t21, inf), 2688)],
        [(jnp.where(lane >= 48, t21, inf), 2688)]
        + [(c1[:, t * 128:(t + 1) * 128], 2816 + t * 128) for t in range(20)]
        + [(jnp.where(lane < 96, t42, inf), 5376)],
        [(jnp.where(lane >= 96, t42, inf), 5376)]
        + [(c2[:, t * 128:(t + 1) * 128], 5504 + t * 128) for t in range(21)],
    )

    acc = jnp.full((bt, 1), jnp.inf, jnp.float32)
    idx = jnp.zeros((bt, 1), jnp.int32)
    for tiles in win_tiles:
        lmin = tiles[0][0]
        for v, _ in tiles[1:]:
            lmin = jnp.minimum(lmin, v)
        m = jnp.min(lmin, axis=1, keepdims=True)
        cand = None
        for v, base in tiles:
            ci = jnp.where(v == m, base + lane, big)
            cand = ci if cand is None else jnp.minimum(cand, ci)
        i = jnp.min(cand, axis=1, keepdims=True)
        take = m < acc
        idx = jnp.where(take, i, idx)
        acc = jnp.where(take, m.astype(jnp.bfloat16).astype(jnp.float32), acc)
    idx_ref[0, 0, :] = idx[:, 0]


def _st_body(xn_ref, q_ref, qst_ref, loss_ref, acc_ref):
    i = pl.program_id(0)
    xn = xn_ref[...]
    d = q_ref[...] - xn
    qst_ref[...] = xn + d

    @pl.when(i == 0)
    def _():
        acc_ref[0] = 0.0

    acc_ref[0] += jnp.sum(d * d)

    @pl.when(i == NBLK - 1)
    def _():
        loss_ref[...] = jnp.reshape(
            acc_ref[0] * jnp.float32(1.25 / (N_TOK * DIM)), (1, 1))


@functools.cache
def _make_sc_gather():
    @functools.partial(
        pl.kernel,
        out_type=jax.ShapeDtypeStruct((N_TOK, DIM), jnp.float32),
        mesh=plsc.VectorSubcoreMesh(core_axis_name="c", subcore_axis_name="s"),
        scratch_types=[
            pltpu.VMEM((_B_PER_W,), jnp.int32),
            pltpu.VMEM((_B_PER_W, DIM), jnp.float32),
            pltpu.SemaphoreType.DMA,
        ],
    )
    def _sc_gather(table_hbm, idx_hbm, out_hbm, idx_v, rows_v, sem):
        wid = lax.axis_index("s") * 2 + lax.axis_index("c")
        base = wid * _B_PER_W
        pltpu.sync_copy(idx_hbm.at[pl.ds(base, _B_PER_W)], idx_v)
        pltpu.async_copy(table_hbm.at[idx_v], rows_v, sem).wait()
        pltpu.sync_copy(rows_v, out_hbm.at[pl.ds(base, _B_PER_W)])

    return _sc_gather


def kernel(x, codebook):
    xn, wn, idx3 = pl.pallas_call(
        _dist_body,
        grid=(NBLK,),
        in_specs=[
            pl.BlockSpec((BT, DIM), lambda i: (i, 0)),
            pl.BlockSpec((N_EMB, DIM), lambda i: (0, 0)),
        ],
        out_specs=[
            pl.BlockSpec((BT, DIM), lambda i: (i, 0)),
            pl.BlockSpec((N_EMB, DIM), lambda i: (0, 0)),
            pl.BlockSpec((1, 1, BT), lambda i: (i, 0, 0)),
        ],
        out_shape=[
            jax.ShapeDtypeStruct((N_TOK, DIM), jnp.float32),
            jax.ShapeDtypeStruct((N_EMB, DIM), jnp.float32),
            jax.ShapeDtypeStruct((NBLK, 1, BT), jnp.int32),
        ],
        scratch_shapes=[pltpu.VMEM((1, N_EMB), jnp.float32),
                        pltpu.VMEM((N_EMB, DIM), jnp.bfloat16)],
    )(x, codebook)
    idx = idx3.reshape(N_TOK)
    q = _make_sc_gather()(wn, idx)
    qst, loss = pl.pallas_call(
        _st_body,
        grid=(NBLK,),
        in_specs=[
            pl.BlockSpec((BT, DIM), lambda i: (i, 0)),
            pl.BlockSpec((BT, DIM), lambda i: (i, 0)),
        ],
        out_specs=[
            pl.BlockSpec((BT, DIM), lambda i: (i, 0)),
            pl.BlockSpec((1, 1), lambda i: (0, 0)),
        ],
        out_shape=[
            jax.ShapeDtypeStruct((N_TOK, DIM), jnp.float32),
            jax.ShapeDtypeStruct((1, 1), jnp.float32),
        ],
        scratch_shapes=[pltpu.SMEM((1,), jnp.float32)],
    )(xn, q)
    return qst, loss[0, 0], idx


# speedup vs baseline: 1.4441x; 1.1248x over previous
import functools

import jax
import jax.numpy as jnp
from jax import lax
from jax.experimental import pallas as pl
from jax.experimental.pallas import tpu as pltpu
from jax.experimental.pallas import tpu_sc as plsc

N_TOK = 9216
N_EMB = 8192
DIM = 256
BT = 512
NBLK = N_TOK // BT
EPS = 1e-12
_CHUNK = 2736

_SC_NW = 32
_B_PER_W = N_TOK // _SC_NW


def _norm_rows(v):
    n = jnp.sqrt(jnp.sum(jnp.abs(v) ** 2, axis=1, keepdims=True))
    return v / jnp.maximum(n, EPS)


def _dist_body(x_ref, cb_ref, xn_ref, wn_ref, idx_ref, wn2_ref, wnbf_ref):
    i = pl.program_id(0)

    @pl.when(i == 0)
    def _():
        wn = _norm_rows(cb_ref[...])
        wn_ref[...] = wn
        wn2_ref[...] = jnp.sum(wn ** 2, axis=1)[None, :]
        wnbf_ref[...] = wn.astype(jnp.bfloat16)

    xn = _norm_rows(x_ref[...])
    xn_ref[...] = xn
    xn2 = jnp.sum(xn ** 2, axis=1, keepdims=True)
    bt = xn.shape[0]
    xm2b = (-2.0 * xn).astype(jnp.bfloat16)
    inf = jnp.float32(jnp.inf)
    big = jnp.int32(N_EMB)
    lane = lax.broadcasted_iota(jnp.int32, (bt, 128), 1)

    def dchunk(lo, hi):
        dots = lax.dot_general(xm2b, wnbf_ref[lo:hi, :],
                               (((1,), (1,)), ((), ())),
                               preferred_element_type=jnp.float32)
        return (xn2 + wn2_ref[:, lo:hi]) + dots

    c0 = dchunk(0, 2816)
    c1 = dchunk(2816, 5504)
    c2 = dchunk(5504, 8192)

    t21 = c0[:, 2688:2816]
    t42 = c1[:, 2560:2688]
    win_tiles = (
        [(c0[:, t * 128:(t + 1) * 128], t * 128) for t in range(21)]
        + [(jnp.where(lane < 48, t21, inf), 2688)],
        [(jnp.where(lane >= 48, t21, inf), 2688)]
        + [(c1[:, t * 128:(t + 1) * 128], 2816 + t * 128) for t in range(20)]
        + [(jnp.where(lane < 96, t42, inf), 5376)],
        [(jnp.where(lane >= 96, t42, inf), 5376)]
        + [(c2[:, t * 128:(t + 1) * 128], 5504 + t * 128) for t in range(21)],
    )

    acc = jnp.full((bt, 1), jnp.inf, jnp.float32)
    idx = jnp.zeros((bt, 1), jnp.int32)
    for tiles in win_tiles:
        lmin = tiles[0][0]
        for v, _ in tiles[1:]:
            lmin = jnp.minimum(lmin, v)
        m = jnp.min(lmin, axis=1, keepdims=True)
        cand = None
        for v, base in tiles:
            ci = jnp.where(v == m, base + lane, big)
            cand = ci if cand is None else jnp.minimum(cand, ci)
        i = jnp.min(cand, axis=1, keepdims=True)
        take = m < acc
        idx = jnp.where(take, i, idx)
        acc = jnp.where(take, m.astype(jnp.bfloat16).astype(jnp.float32), acc)
    idx_ref[0, 0, :] = idx[:, 0]


def _st_body(xn_ref, q_ref, qst_ref, loss_ref, acc_ref):
    i = pl.program_id(0)
    xn = xn_ref[...]
    d = q_ref[...] - xn
    qst_ref[...] = xn + d

    @pl.when(i == 0)
    def _():
        acc_ref[0] = 0.0

    acc_ref[0] += jnp.sum(d * d)

    @pl.when(i == NBLK - 1)
    def _():
        loss_ref[...] = jnp.reshape(
            acc_ref[0] * jnp.float32(1.25 / (N_TOK * DIM)), (1, 1))


@functools.cache
def _make_sc_gather():
    @functools.partial(
        pl.kernel,
        out_type=jax.ShapeDtypeStruct((N_TOK, DIM), jnp.float32),
        mesh=plsc.VectorSubcoreMesh(core_axis_name="c", subcore_axis_name="s"),
        scratch_types=[
            pltpu.VMEM((_B_PER_W,), jnp.int32),
            pltpu.VMEM((_B_PER_W, DIM), jnp.float32),
            pltpu.SemaphoreType.DMA,
        ],
    )
    def _sc_gather(table_hbm, idx_hbm, out_hbm, idx_v, rows_v, sem):
        wid = lax.axis_index("s") * 2 + lax.axis_index("c")
        base = wid * _B_PER_W
        pltpu.sync_copy(idx_hbm.at[pl.ds(base, _B_PER_W)], idx_v)
        pltpu.async_copy(table_hbm.at[idx_v], rows_v, sem).wait()
        pltpu.sync_copy(rows_v, out_hbm.at[pl.ds(base, _B_PER_W)])

    return _sc_gather


def kernel(x, codebook):
    xn, wn, idx3 = pl.pallas_call(
        _dist_body,
        grid=(NBLK,),
        in_specs=[
            pl.BlockSpec((BT, DIM), lambda i: (i, 0)),
            pl.BlockSpec((N_EMB, DIM), lambda i: (0, 0)),
        ],
        out_specs=[
            pl.BlockSpec((BT, DIM), lambda i: (i, 0)),
            pl.BlockSpec((N_EMB, DIM), lambda i: (0, 0)),
            pl.BlockSpec((1, 1, BT), lambda i: (i, 0, 0)),
        ],
        out_shape=[
            jax.ShapeDtypeStruct((N_TOK, DIM), jnp.float32),
            jax.ShapeDtypeStruct((N_EMB, DIM), jnp.float32),
            jax.ShapeDtypeStruct((NBLK, 1, BT), jnp.int32),
        ],
        scratch_shapes=[pltpu.VMEM((1, N_EMB), jnp.float32),
                        pltpu.VMEM((N_EMB, DIM), jnp.bfloat16)],
    )(x, codebook)
    idx = idx3.reshape(N_TOK)
    q = _make_sc_gather()(wn, idx)
    qst, loss = pl.pallas_call(
        _st_body,
        grid=(NBLK,),
        in_specs=[
            pl.BlockSpec((BT, DIM), lambda i: (i, 0)),
            pl.BlockSpec((BT, DIM), lambda i: (i, 0)),
        ],
        out_specs=[
            pl.BlockSpec((BT, DIM), lambda i: (i, 0)),
            pl.BlockSpec((1, 1), lambda i: (0, 0)),
        ],
        out_shape=[
            jax.ShapeDtypeStruct((N_TOK, DIM), jnp.float32),
            jax.ShapeDtypeStruct((1, 1), jnp.float32),
        ],
        scratch_shapes=[pltpu.SMEM((1,), jnp.float32)],
    )(xn, q)
    return qst, loss[0, 0], idx


# speedup vs baseline: 1.4898x; 1.0316x over previous
import functools

import jax
import jax.numpy as jnp
from jax import lax
from jax.experimental import pallas as pl
from jax.experimental.pallas import tpu as pltpu
from jax.experimental.pallas import tpu_sc as plsc

N_TOK = 9216
N_EMB = 8192
DIM = 256
BT = 1024
NBLK = N_TOK // BT
EPS = 1e-12
_CHUNK = 2736

_SC_NW = 32
_B_PER_W = N_TOK // _SC_NW


def _norm_rows(v):
    n = jnp.sqrt(jnp.sum(jnp.abs(v) ** 2, axis=1, keepdims=True))
    return v / jnp.maximum(n, EPS)


def _dist_body(x_ref, cb_ref, xn_ref, wn_ref, idx_ref, wn2_ref, wnbf_ref):
    i = pl.program_id(0)

    @pl.when(i == 0)
    def _():
        wn = _norm_rows(cb_ref[...])
        wn_ref[...] = wn
        wn2_ref[...] = jnp.sum(wn ** 2, axis=1)[None, :]
        wnbf_ref[...] = wn.astype(jnp.bfloat16)

    xn = _norm_rows(x_ref[...])
    xn_ref[...] = xn
    xn2 = jnp.sum(xn ** 2, axis=1, keepdims=True)
    bt = xn.shape[0]
    xm2b = (-2.0 * xn).astype(jnp.bfloat16)
    inf = jnp.float32(jnp.inf)
    big = jnp.int32(N_EMB)
    lane = lax.broadcasted_iota(jnp.int32, (bt, 128), 1)

    def dchunk(lo, hi):
        dots = lax.dot_general(xm2b, wnbf_ref[lo:hi, :],
                               (((1,), (1,)), ((), ())),
                               preferred_element_type=jnp.float32)
        return (xn2 + wn2_ref[:, lo:hi]) + dots

    c0 = dchunk(0, 2816)
    c1 = dchunk(2816, 5504)
    c2 = dchunk(5504, 8192)

    t21 = c0[:, 2688:2816]
    t42 = c1[:, 2560:2688]
    win_tiles = (
        [(c0[:, t * 128:(t + 1) * 128], t * 128) for t in range(21)]
        + [(jnp.where(lane < 48, t21, inf), 2688)],
        [(jnp.where(lane >= 48, t21, inf), 2688)]
        + [(c1[:, t * 128:(t + 1) * 128], 2816 + t * 128) for t in range(20)]
        + [(jnp.where(lane < 96, t42, inf), 5376)],
        [(jnp.where(lane >= 96, t42, inf), 5376)]
        + [(c2[:, t * 128:(t + 1) * 128], 5504 + t * 128) for t in range(21)],
    )

    acc = jnp.full((bt, 1), jnp.inf, jnp.float32)
    idx = jnp.zeros((bt, 1), jnp.int32)
    for tiles in win_tiles:
        lmin = tiles[0][0]
        for v, _ in tiles[1:]:
            lmin = jnp.minimum(lmin, v)
        m = jnp.min(lmin, axis=1, keepdims=True)
        cand = None
        for v, base in tiles:
            ci = jnp.where(v == m, base + lane, big)
            cand = ci if cand is None else jnp.minimum(cand, ci)
        i = jnp.min(cand, axis=1, keepdims=True)
        take = m < acc
        idx = jnp.where(take, i, idx)
        acc = jnp.where(take, m.astype(jnp.bfloat16).astype(jnp.float32), acc)
    idx_ref[0, 0, :] = idx[:, 0]


def _st_body(xn_ref, q_ref, qst_ref, loss_ref, acc_ref):
    i = pl.program_id(0)
    xn = xn_ref[...]
    d = q_ref[...] - xn
    qst_ref[...] = xn + d

    @pl.when(i == 0)
    def _():
        acc_ref[0] = 0.0

    acc_ref[0] += jnp.sum(d * d)

    @pl.when(i == NBLK - 1)
    def _():
        loss_ref[...] = jnp.reshape(
            acc_ref[0] * jnp.float32(1.25 / (N_TOK * DIM)), (1, 1))


@functools.cache
def _make_sc_gather():
    @functools.partial(
        pl.kernel,
        out_type=jax.ShapeDtypeStruct((N_TOK, DIM), jnp.float32),
        mesh=plsc.VectorSubcoreMesh(core_axis_name="c", subcore_axis_name="s"),
        scratch_types=[
            pltpu.VMEM((_B_PER_W,), jnp.int32),
            pltpu.VMEM((_B_PER_W, DIM), jnp.float32),
            pltpu.SemaphoreType.DMA,
        ],
    )
    def _sc_gather(table_hbm, idx_hbm, out_hbm, idx_v, rows_v, sem):
        wid = lax.axis_index("s") * 2 + lax.axis_index("c")
        base = wid * _B_PER_W
        pltpu.sync_copy(idx_hbm.at[pl.ds(base, _B_PER_W)], idx_v)
        pltpu.async_copy(table_hbm.at[idx_v], rows_v, sem).wait()
        pltpu.sync_copy(rows_v, out_hbm.at[pl.ds(base, _B_PER_W)])

    return _sc_gather


def kernel(x, codebook):
    xn, wn, idx3 = pl.pallas_call(
        _dist_body,
        grid=(NBLK,),
        in_specs=[
            pl.BlockSpec((BT, DIM), lambda i: (i, 0)),
            pl.BlockSpec((N_EMB, DIM), lambda i: (0, 0)),
        ],
        out_specs=[
            pl.BlockSpec((BT, DIM), lambda i: (i, 0)),
            pl.BlockSpec((N_EMB, DIM), lambda i: (0, 0)),
            pl.BlockSpec((1, 1, BT), lambda i: (i, 0, 0)),
        ],
        out_shape=[
            jax.ShapeDtypeStruct((N_TOK, DIM), jnp.float32),
            jax.ShapeDtypeStruct((N_EMB, DIM), jnp.float32),
            jax.ShapeDtypeStruct((NBLK, 1, BT), jnp.int32),
        ],
        scratch_shapes=[pltpu.VMEM((1, N_EMB), jnp.float32),
                        pltpu.VMEM((N_EMB, DIM), jnp.bfloat16)],
    )(x, codebook)
    idx = idx3.reshape(N_TOK)
    q = _make_sc_gather()(wn, idx)
    qst, loss = pl.pallas_call(
        _st_body,
        grid=(NBLK,),
        in_specs=[
            pl.BlockSpec((BT, DIM), lambda i: (i, 0)),
            pl.BlockSpec((BT, DIM), lambda i: (i, 0)),
        ],
        out_specs=[
            pl.BlockSpec((BT, DIM), lambda i: (i, 0)),
            pl.BlockSpec((1, 1), lambda i: (0, 0)),
        ],
        out_shape=[
            jax.ShapeDtypeStruct((N_TOK, DIM), jnp.float32),
            jax.ShapeDtypeStruct((1, 1), jnp.float32),
        ],
        scratch_shapes=[pltpu.SMEM((1,), jnp.float32)],
    )(xn, q)
    return qst, loss[0, 0], idx


# speedup vs baseline: 1.5222x; 1.0217x over previous
import functools

import jax
import jax.numpy as jnp
from jax import lax
from jax.experimental import pallas as pl
from jax.experimental.pallas import tpu as pltpu
from jax.experimental.pallas import tpu_sc as plsc

N_TOK = 9216
N_EMB = 8192
DIM = 256
BT = 1536
NBLK = N_TOK // BT
EPS = 1e-12
_CHUNK = 2736

_SC_NW = 32
_B_PER_W = N_TOK // _SC_NW


def _norm_rows(v):
    n = jnp.sqrt(jnp.sum(jnp.abs(v) ** 2, axis=1, keepdims=True))
    return v / jnp.maximum(n, EPS)


def _dist_body(x_ref, cb_ref, xn_ref, wn_ref, idx_ref, wn2_ref, wnbf_ref):
    i = pl.program_id(0)

    @pl.when(i == 0)
    def _():
        wn = _norm_rows(cb_ref[...])
        wn_ref[...] = wn
        wn2_ref[...] = jnp.sum(wn ** 2, axis=1)[None, :]
        wnbf_ref[...] = wn.astype(jnp.bfloat16)

    xn = _norm_rows(x_ref[...])
    xn_ref[...] = xn
    xn2 = jnp.sum(xn ** 2, axis=1, keepdims=True)
    bt = xn.shape[0]
    xm2b = (-2.0 * xn).astype(jnp.bfloat16)
    inf = jnp.float32(jnp.inf)
    big = jnp.int32(N_EMB)
    lane = lax.broadcasted_iota(jnp.int32, (bt, 128), 1)

    def dchunk(lo, hi):
        dots = lax.dot_general(xm2b, wnbf_ref[lo:hi, :],
                               (((1,), (1,)), ((), ())),
                               preferred_element_type=jnp.float32)
        return (xn2 + wn2_ref[:, lo:hi]) + dots

    c0 = dchunk(0, 2816)
    c1 = dchunk(2816, 5504)
    c2 = dchunk(5504, 8192)

    t21 = c0[:, 2688:2816]
    t42 = c1[:, 2560:2688]
    win_tiles = (
        [(c0[:, t * 128:(t + 1) * 128], t * 128) for t in range(21)]
        + [(jnp.where(lane < 48, t21, inf), 2688)],
        [(jnp.where(lane >= 48, t21, inf), 2688)]
        + [(c1[:, t * 128:(t + 1) * 128], 2816 + t * 128) for t in range(20)]
        + [(jnp.where(lane < 96, t42, inf), 5376)],
        [(jnp.where(lane >= 96, t42, inf), 5376)]
        + [(c2[:, t * 128:(t + 1) * 128], 5504 + t * 128) for t in range(21)],
    )

    acc = jnp.full((bt, 1), jnp.inf, jnp.float32)
    idx = jnp.zeros((bt, 1), jnp.int32)
    for tiles in win_tiles:
        lmin = tiles[0][0]
        for v, _ in tiles[1:]:
            lmin = jnp.minimum(lmin, v)
        m = jnp.min(lmin, axis=1, keepdims=True)
        cand = None
        for v, base in tiles:
            ci = jnp.where(v == m, base + lane, big)
            cand = ci if cand is None else jnp.minimum(cand, ci)
        i = jnp.min(cand, axis=1, keepdims=True)
        take = m < acc
        idx = jnp.where(take, i, idx)
        acc = jnp.where(take, m.astype(jnp.bfloat16).astype(jnp.float32), acc)
    idx_ref[0, 0, :] = idx[:, 0]


def _st_body(xn_ref, q_ref, qst_ref, loss_ref, acc_ref):
    i = pl.program_id(0)
    xn = xn_ref[...]
    d = q_ref[...] - xn
    qst_ref[...] = xn + d

    @pl.when(i == 0)
    def _():
        acc_ref[0] = 0.0

    acc_ref[0] += jnp.sum(d * d)

    @pl.when(i == NBLK - 1)
    def _():
        loss_ref[...] = jnp.reshape(
            acc_ref[0] * jnp.float32(1.25 / (N_TOK * DIM)), (1, 1))


@functools.cache
def _make_sc_gather():
    @functools.partial(
        pl.kernel,
        out_type=jax.ShapeDtypeStruct((N_TOK, DIM), jnp.float32),
        mesh=plsc.VectorSubcoreMesh(core_axis_name="c", subcore_axis_name="s"),
        scratch_types=[
            pltpu.VMEM((_B_PER_W,), jnp.int32),
            pltpu.VMEM((_B_PER_W, DIM), jnp.float32),
            pltpu.SemaphoreType.DMA,
        ],
    )
    def _sc_gather(table_hbm, idx_hbm, out_hbm, idx_v, rows_v, sem):
        wid = lax.axis_index("s") * 2 + lax.axis_index("c")
        base = wid * _B_PER_W
        pltpu.sync_copy(idx_hbm.at[pl.ds(base, _B_PER_W)], idx_v)
        pltpu.async_copy(table_hbm.at[idx_v], rows_v, sem).wait()
        pltpu.sync_copy(rows_v, out_hbm.at[pl.ds(base, _B_PER_W)])

    return _sc_gather


def kernel(x, codebook):
    xn, wn, idx3 = pl.pallas_call(
        _dist_body,
        grid=(NBLK,),
        in_specs=[
            pl.BlockSpec((BT, DIM), lambda i: (i, 0)),
            pl.BlockSpec((N_EMB, DIM), lambda i: (0, 0)),
        ],
        out_specs=[
            pl.BlockSpec((BT, DIM), lambda i: (i, 0)),
            pl.BlockSpec((N_EMB, DIM), lambda i: (0, 0)),
            pl.BlockSpec((1, 1, BT), lambda i: (i, 0, 0)),
        ],
        out_shape=[
            jax.ShapeDtypeStruct((N_TOK, DIM), jnp.float32),
            jax.ShapeDtypeStruct((N_EMB, DIM), jnp.float32),
            jax.ShapeDtypeStruct((NBLK, 1, BT), jnp.int32),
        ],
        scratch_shapes=[pltpu.VMEM((1, N_EMB), jnp.float32),
                        pltpu.VMEM((N_EMB, DIM), jnp.bfloat16)],
    )(x, codebook)
    idx = idx3.reshape(N_TOK)
    q = _make_sc_gather()(wn, idx)
    qst, loss = pl.pallas_call(
        _st_body,
        grid=(NBLK,),
        in_specs=[
            pl.BlockSpec((BT, DIM), lambda i: (i, 0)),
            pl.BlockSpec((BT, DIM), lambda i: (i, 0)),
        ],
        out_specs=[
            pl.BlockSpec((BT, DIM), lambda i: (i, 0)),
            pl.BlockSpec((1, 1), lambda i: (0, 0)),
        ],
        out_shape=[
            jax.ShapeDtypeStruct((N_TOK, DIM), jnp.float32),
            jax.ShapeDtypeStruct((1, 1), jnp.float32),
        ],
        scratch_shapes=[pltpu.SMEM((1,), jnp.float32)],
    )(xn, q)
    return qst, loss[0, 0], idx
